# baseline (speedup 1.0000x reference)
import jax
import jax.numpy as jnp
from jax.experimental import pallas as pl


def _add_body(patch_ref, table_ref, out_ref):
    out_ref[...] = patch_ref[...] + table_ref[...]


def kernel(patch, pos_table):
    B, P, D = patch.shape
    bb = 4
    return pl.pallas_call(
        _add_body,
        grid=(B // bb,),
        in_specs=[
            pl.BlockSpec((bb, P, D), lambda i: (i, 0, 0)),
            pl.BlockSpec((P, D), lambda i: (0, 0)),
        ],
        out_specs=pl.BlockSpec((bb, P, D), lambda i: (i, 0, 0)),
        out_shape=jax.ShapeDtypeStruct((B, P, D), patch.dtype),
    )(patch, pos_table)


# speedup vs baseline: 1.0044x; 1.0044x over previous
import jax
import jax.numpy as jnp
from jax.experimental import pallas as pl


def _add_body(patch_ref, table_ref, out_ref):
    out_ref[...] = patch_ref[...] + table_ref[...]


def kernel(patch, pos_table):
    B, P, D = patch.shape
    bb = 8
    return pl.pallas_call(
        _add_body,
        grid=(B // bb,),
        in_specs=[
            pl.BlockSpec((bb, P, D), lambda i: (i, 0, 0)),
            pl.BlockSpec((P, D), lambda i: (0, 0)),
        ],
        out_specs=pl.BlockSpec((bb, P, D), lambda i: (i, 0, 0)),
        out_shape=jax.ShapeDtypeStruct((B, P, D), patch.dtype),
    )(patch, pos_table)
